# baseline (speedup 1.0000x reference)
import functools

import jax
import jax.numpy as jnp
from jax import lax
from jax.experimental import pallas as pl
from jax.experimental.pallas import tpu as pltpu
from jax.experimental.pallas import tpu_sc as plsc

D_MODEL = 1024
PE_DIM = 256

NC = 2
NS = 16
NW = NC * NS

P_STEP = 48
P_WORKER = 1056
STEPS = P_WORKER // P_STEP


def _sc_gather(idx_flat, pe2d, total_pos):
    mesh = plsc.VectorSubcoreMesh(core_axis_name="c", subcore_axis_name="s")

    @functools.partial(
        pl.kernel,
        out_type=jax.ShapeDtypeStruct((total_pos * 4, PE_DIM), jnp.float32),
        mesh=mesh,
        scratch_types=[
            pltpu.VMEM((4 * P_STEP,), jnp.int32),
            pltpu.VMEM((4 * P_STEP, PE_DIM), jnp.float32),
            pltpu.SemaphoreType.DMA,
        ],
    )
    def k(idx_hbm, pe_hbm, ie_hbm, idx_v, rows_v, gsem):
        wid = lax.axis_index("s") * NC + lax.axis_index("c")
        start = jnp.minimum(wid * P_WORKER, total_pos - P_WORKER)

        def body(s, carry):
            r0 = (start + s * P_STEP) * 4
            pltpu.sync_copy(idx_hbm.at[pl.ds(r0, 4 * P_STEP)], idx_v)
            pltpu.async_copy(pe_hbm.at[idx_v], rows_v, gsem).wait()
            pltpu.sync_copy(rows_v, ie_hbm.at[pl.ds(r0, 4 * P_STEP)])
            return carry

        lax.fori_loop(0, STEPS, body, 0)

    return k(idx_flat, pe2d)


def _tc_add(seq2d, ie2d):
    rows = seq2d.shape[0]
    br = 1024

    def body(a_ref, b_ref, o_ref):
        o_ref[...] = a_ref[...] + b_ref[...]

    return pl.pallas_call(
        body,
        grid=(pl.cdiv(rows, br),),
        in_specs=[
            pl.BlockSpec((br, D_MODEL), lambda i: (i, 0)),
            pl.BlockSpec((br, D_MODEL), lambda i: (i, 0)),
        ],
        out_specs=pl.BlockSpec((br, D_MODEL), lambda i: (i, 0)),
        out_shape=jax.ShapeDtypeStruct((rows, D_MODEL), jnp.float32),
    )(seq2d, ie2d)


def kernel(seq, coords, seq_id, pe):
    B, L1, D = seq.shape
    total_pos = B * L1
    idx_body = jnp.concatenate([coords, seq_id[:, :, None]], axis=-1)
    idx = jnp.concatenate(
        [jnp.zeros((B, 1, 4), jnp.int32), idx_body], axis=1)
    idx_flat = idx.reshape(-1)

    ie = _sc_gather(idx_flat, pe[0], total_pos)
    ie2d = ie.reshape(total_pos, D)
    out2d = _tc_add(seq.reshape(total_pos, D), ie2d)
    return (out2d.reshape(B, L1, D), ie2d.reshape(B, L1, D))


# speedup vs baseline: 1.6504x; 1.6504x over previous
import functools

import jax
import jax.numpy as jnp
from jax import lax
from jax.experimental import pallas as pl
from jax.experimental.pallas import tpu as pltpu
from jax.experimental.pallas import tpu_sc as plsc

D_MODEL = 1024
PE_DIM = 256
GROUPS = D_MODEL // 16

NC = 2
NS = 16
NW = NC * NS

P = 16
P_WORKER = 1024
S = P_WORKER // P
TAIL = 4


def _sc_fused(idx_flat, pe2d, seq2d, total_pos):
    mesh = plsc.VectorSubcoreMesh(core_axis_name="c", subcore_axis_name="s")

    @functools.partial(
        pl.kernel,
        out_type=[
            jax.ShapeDtypeStruct((total_pos, D_MODEL), jnp.float32),
            jax.ShapeDtypeStruct((total_pos, D_MODEL), jnp.float32),
        ],
        mesh=mesh,
        scratch_types=[
            pltpu.VMEM((2, 4 * P), jnp.int32),
            pltpu.VMEM((2, 4 * P, PE_DIM), jnp.float32),
            pltpu.VMEM((2, P, D_MODEL), jnp.float32),
            pltpu.VMEM((2, P, D_MODEL), jnp.float32),
            pltpu.VMEM((4 * TAIL,), jnp.int32),
            pltpu.VMEM((4 * TAIL, PE_DIM), jnp.float32),
            pltpu.VMEM((TAIL, D_MODEL), jnp.float32),
            pltpu.VMEM((TAIL, D_MODEL), jnp.float32),
            pltpu.SemaphoreType.DMA,
            pltpu.SemaphoreType.DMA,
            pltpu.SemaphoreType.DMA,
            pltpu.SemaphoreType.DMA,
            pltpu.SemaphoreType.DMA,
            pltpu.SemaphoreType.DMA,
            pltpu.SemaphoreType.DMA,
            pltpu.SemaphoreType.DMA,
        ],
    )
    def k(idx_hbm, pe_hbm, seq_hbm, out_hbm, ie_hbm,
          idx_v, rows_v, outv, iev,
          t_idx, t_rows, t_out, t_ie,
          g0, g1, s0, s1, o0, o1, e0, e1):
        wid = lax.axis_index("s") * NC + lax.axis_index("c")
        start = wid * P_WORKER
        gsem = (g0, g1)
        ssem = (s0, s1)
        osem = (o0, o1)
        esem = (e0, e1)

        def load_step(s, b):
            p0 = start + s * P
            pltpu.sync_copy(idx_hbm.at[pl.ds(p0 * 4, 4 * P)], idx_v.at[b])
            gd = pltpu.async_copy(pe_hbm.at[idx_v.at[b]], rows_v.at[b],
                                  gsem[b])
            sd = pltpu.async_copy(seq_hbm.at[pl.ds(p0, P)], outv.at[b],
                                  ssem[b])
            return gd, sd

        def compute(b):
            def body(p, carry):
                for j in range(GROUPS):
                    sl = pl.ds(j * 16, 16)
                    sv = outv[b, p, sl]
                    gv = rows_v[b, 4 * p + j // 16, pl.ds((j % 16) * 16, 16)]
                    iev[b, p, sl] = gv
                    outv[b, p, sl] = sv + gv
                return carry
            lax.fori_loop(0, P, body, 0)

        def write_step(s, b):
            p0 = start + s * P
            pltpu.async_copy(outv.at[b], out_hbm.at[pl.ds(p0, P)], osem[b])
            pltpu.async_copy(iev.at[b], ie_hbm.at[pl.ds(p0, P)], esem[b])

        def drain_writes(b):
            pltpu.make_async_copy(out_hbm.at[pl.ds(0, P)], outv.at[b],
                                  osem[b]).wait()
            pltpu.make_async_copy(ie_hbm.at[pl.ds(0, P)], iev.at[b],
                                  esem[b]).wait()

        def pair(i, carry):
            st = 2 * i

            @pl.when(i > 0)
            def _():
                drain_writes(0)
            gd0, sd0 = load_step(st, 0)

            @pl.when(i > 0)
            def _():
                drain_writes(1)
            gd1, sd1 = load_step(st + 1, 1)

            gd0.wait()
            sd0.wait()
            compute(0)
            write_step(st, 0)

            gd1.wait()
            sd1.wait()
            compute(1)
            write_step(st + 1, 1)
            return carry

        lax.fori_loop(0, S // 2, pair, 0)
        drain_writes(0)
        drain_writes(1)

        @pl.when(wid == NW - 1)
        def _():
            t0 = NW * P_WORKER
            pltpu.sync_copy(idx_hbm.at[pl.ds(t0 * 4, 4 * TAIL)], t_idx)
            pltpu.async_copy(pe_hbm.at[t_idx], t_rows, g0).wait()
            pltpu.async_copy(seq_hbm.at[pl.ds(t0, TAIL)], t_out, s0).wait()

            def tbody(p, carry):
                for j in range(GROUPS):
                    sl = pl.ds(j * 16, 16)
                    sv = t_out[p, sl]
                    gv = t_rows[4 * p + j // 16, pl.ds((j % 16) * 16, 16)]
                    t_ie[p, sl] = gv
                    t_out[p, sl] = sv + gv
                return carry
            lax.fori_loop(0, TAIL, tbody, 0)
            pltpu.sync_copy(t_out, out_hbm.at[pl.ds(t0, TAIL)])
            pltpu.sync_copy(t_ie, ie_hbm.at[pl.ds(t0, TAIL)])

    return k(idx_flat, pe2d, seq2d)


def kernel(seq, coords, seq_id, pe):
    B, L1, D = seq.shape
    total_pos = B * L1
    idx_body = jnp.concatenate([coords, seq_id[:, :, None]], axis=-1)
    idx = jnp.concatenate(
        [jnp.zeros((B, 1, 4), jnp.int32), idx_body], axis=1)
    idx_flat = idx.reshape(-1)

    out2d, ie2d = _sc_fused(idx_flat, pe[0], seq.reshape(total_pos, D),
                            total_pos)
    return (out2d.reshape(B, L1, D), ie2d.reshape(B, L1, D))
